# baseline (device time: 53093 ns/iter reference)
import jax
import jax.numpy as jnp
from jax import lax
from jax.experimental import pallas as pl
from jax.experimental.pallas import tpu as pltpu

N_DEV = 8
N_HOP = N_DEV - 1
M = 1024
M_HALF = M // 2
N_CHUNK = 512
K_SUB = 4
N_SUB = N_CHUNK // K_SUB


def kernel(x):
    x = x.reshape(M, N_DEV * N_CHUNK)

    def body(
        x_ref,
        out_ref,
        xr,
        comm_a,
        comm_b,
        acc_a,
        acc_b,
        send_sems_a,
        send_sems_b,
        recv_sems_a,
        recv_sems_b,
    ):
        my = lax.axis_index("i")
        left = lax.rem(my + N_DEV - 1, N_DEV)
        right = lax.rem(my + 1, N_DEV)

        barrier_sem = pltpu.get_barrier_semaphore()
        for nbr in (left, right):
            pl.semaphore_signal(
                barrier_sem, inc=1,
                device_id=(nbr,), device_id_type=pl.DeviceIdType.MESH,
            )
        pl.semaphore_wait(barrier_sem, 2)

        def ksl(k):
            return pl.ds(k * N_SUB, N_SUB)

        def stage(j):
            c = lax.rem(my + j, N_DEV)
            xr[j] = x_ref[:, pl.ds(c * N_CHUNK, N_CHUNK)].astype(jnp.bfloat16)

        def send(dir_tag, h, k):
            comm, ssems, rsems, tgt = (
                (comm_a, send_sems_a, recv_sems_a, right)
                if dir_tag == 0
                else (comm_b, send_sems_b, recv_sems_b, left)
            )
            rows = slice(0, M_HALF) if dir_tag == 0 else slice(M_HALF, M)
            if h == 0:
                src = xr.at[7 if dir_tag == 0 else 1, rows, ksl(k)]
            else:
                src = (acc_a if dir_tag == 0 else acc_b).at[k]
            return pltpu.make_async_remote_copy(
                src_ref=src,
                dst_ref=comm.at[h, k],
                send_sem=ssems.at[k],
                recv_sem=rsems.at[h, k],
                device_id=(tgt,),
                device_id_type=pl.DeviceIdType.MESH,
            )

        stage(7)
        stage(1)
        for k in range(K_SUB):
            send(0, 0, k).start()
            send(1, 0, k).start()
        for j in (6, 2, 5, 3, 4, 0):
            stage(j)

        for h in range(N_HOP):
            ja = 6 - h
            jb = (2 + h) % N_DEV
            for k in range(K_SUB):
                send(0, h, k).wait_recv()
                if h < N_HOP - 1:
                    send(0, h, k).wait_send()
                    acc_a[k] = comm_a[h, k] + xr[ja, :M_HALF, ksl(k)]
                    send(0, h + 1, k).start()
                else:
                    out_ref[:M_HALF, ksl(k)] = (
                        comm_a[h, k] + xr[ja, :M_HALF, ksl(k)]
                    )
                send(1, h, k).wait_recv()
                if h < N_HOP - 1:
                    send(1, h, k).wait_send()
                    acc_b[k] = comm_b[h, k] + xr[jb, M_HALF:, ksl(k)]
                    send(1, h + 1, k).start()
                else:
                    out_ref[M_HALF:, ksl(k)] = (
                        comm_b[h, k] + xr[jb, M_HALF:, ksl(k)]
                    )

        for k in range(K_SUB):
            send(0, N_HOP - 1, k).wait_send()
            send(1, N_HOP - 1, k).wait_send()

    return pl.pallas_call(
        body,
        out_shape=jax.ShapeDtypeStruct((M, N_CHUNK), jnp.bfloat16),
        in_specs=[pl.BlockSpec(memory_space=pltpu.VMEM)],
        out_specs=pl.BlockSpec(memory_space=pltpu.VMEM),
        scratch_shapes=[
            pltpu.VMEM((N_DEV, M, N_CHUNK), jnp.bfloat16),
            pltpu.VMEM((N_HOP, K_SUB, M_HALF, N_SUB), jnp.bfloat16),
            pltpu.VMEM((N_HOP, K_SUB, M_HALF, N_SUB), jnp.bfloat16),
            pltpu.VMEM((K_SUB, M_HALF, N_SUB), jnp.bfloat16),
            pltpu.VMEM((K_SUB, M_HALF, N_SUB), jnp.bfloat16),
            pltpu.SemaphoreType.DMA((K_SUB,)),
            pltpu.SemaphoreType.DMA((K_SUB,)),
            pltpu.SemaphoreType.DMA((N_HOP, K_SUB)),
            pltpu.SemaphoreType.DMA((N_HOP, K_SUB)),
        ],
        compiler_params=pltpu.CompilerParams(collective_id=0),
    )(x)
